# baseline (speedup 1.0000x reference)
import functools

import jax
import jax.numpy as jnp
from jax.experimental import pallas as pl
from jax.experimental.pallas import tpu as pltpu
from jax.sharding import Mesh, PartitionSpec as P

try:
    from jax.experimental.shard_map import shard_map as _shard_map
except ImportError:
    _shard_map = jax.shard_map


def _sigmoid(x):
    return 0.5 * jnp.tanh(0.5 * x) + 0.5


def _ffn_body(x_ref, w13_ref, w2_ref, rms_ref, wf_ref, d_ref, wfb_ref):
    F = w2_ref.shape[0]
    x = x_ref[...].astype(jnp.bfloat16)
    a = jnp.dot(x, w13_ref[:, :F], preferred_element_type=jnp.float32)
    c = jnp.dot(x, w13_ref[:, F:], preferred_element_type=jnp.float32)
    h = (a * _sigmoid(a)) * c
    dec = jnp.dot(h.astype(jnp.bfloat16), w2_ref[...],
                  preferred_element_type=jnp.float32)
    dec = dec * _sigmoid(dec)
    dec = dec * jax.lax.rsqrt(
        jnp.mean(dec * dec, axis=-1, keepdims=True) + 1e-6)
    dec = dec * rms_ref[...]
    d_ref[...] = dec.astype(jnp.bfloat16)
    wfb_ref[...] = wf_ref[...].astype(jnp.bfloat16)


def _out_body(d_ref, wfb_ref, bias_ref, out_ref):
    ob = pl.program_id(1)
    BO = out_ref.shape[1]
    logit = jnp.dot(d_ref[...], wfb_ref[:, pl.ds(ob * BO, BO)],
                    preferred_element_type=jnp.float32)
    logit = logit + bias_ref[:, pl.ds(ob * BO, BO)]
    out_ref[...] = _sigmoid(logit)


def _shard_fn(x, W1, W2, W3, rms2, Wf, bf2):
    Ll, D = x.shape
    w13b = jnp.concatenate([
        jax.lax.all_gather(W1.astype(jnp.bfloat16), "d", axis=1, tiled=True),
        jax.lax.all_gather(W3.astype(jnp.bfloat16), "d", axis=1, tiled=True),
    ], axis=1)
    w2b = jax.lax.all_gather(W2.astype(jnp.bfloat16), "d", axis=0, tiled=True)
    F = w2b.shape[0]
    Ol = Wf.shape[1]

    BL_A = min(256, Ll)
    grid_a = Ll // BL_A
    WFC = Ol // grid_a

    d, wfb = pl.pallas_call(
        _ffn_body,
        grid=(grid_a,),
        in_specs=[
            pl.BlockSpec((BL_A, D), lambda i: (i, 0)),
            pl.BlockSpec((D, 2 * F), lambda i: (0, 0)),
            pl.BlockSpec((F, D), lambda i: (0, 0)),
            pl.BlockSpec((1, D), lambda i: (0, 0)),
            pl.BlockSpec((D, WFC), lambda i: (0, i)),
        ],
        out_specs=[
            pl.BlockSpec((BL_A, D), lambda i: (i, 0)),
            pl.BlockSpec((D, WFC), lambda i: (0, i)),
        ],
        out_shape=[
            jax.ShapeDtypeStruct((Ll, D), jnp.bfloat16),
            jax.ShapeDtypeStruct((D, Ol), jnp.bfloat16),
        ],
        compiler_params=pltpu.CompilerParams(
            dimension_semantics=("arbitrary",),
        ),
    )(x, w13b, w2b, rms2, Wf)

    d_full = jax.lax.all_gather(d, "d", axis=0, tiled=True)
    L = d_full.shape[0]

    BL_B = min(1024, L)
    BO_B = min(2048, Ol)
    out = pl.pallas_call(
        _out_body,
        grid=(L // BL_B, Ol // BO_B),
        in_specs=[
            pl.BlockSpec((BL_B, D), lambda lb, ob: (lb, 0)),
            pl.BlockSpec((D, Ol), lambda lb, ob: (0, 0)),
            pl.BlockSpec((1, Ol), lambda lb, ob: (0, 0)),
        ],
        out_specs=pl.BlockSpec((BL_B, BO_B), lambda lb, ob: (lb, ob)),
        out_shape=jax.ShapeDtypeStruct((L, BO_B * (Ol // BO_B)), jnp.float32),
        compiler_params=pltpu.CompilerParams(
            dimension_semantics=("arbitrary", "arbitrary"),
        ),
    )(d_full, wfb, bf2)
    return out


def _run(x, W1, W2, W3, rms_w, Wf, bf):
    D = x.shape[1]
    O = Wf.shape[1]
    rms2 = rms_w.reshape(1, D)
    bf2 = bf.reshape(1, O)

    devs = jax.devices()
    n = 2 if len(devs) >= 2 else 1
    mesh = Mesh(devs[:n], ("d",))
    fn = _shard_map(
        _shard_fn,
        mesh=mesh,
        in_specs=(P("d", None), P(None, "d"), P("d", None), P(None, "d"),
                  P(None, None), P(None, "d"), P(None, "d")),
        out_specs=P(None, "d"),
        check_rep=False,
    )
    return fn(x, W1, W2, W3, rms2, Wf, bf2)


def kernel(inputs, label, W1, W2, W3, rms_w, Wf, bf):
    del label
    x = inputs[0]
    out = jax.jit(_run)(x, W1, W2, W3, rms_w, Wf, bf)
    return out[None]


# speedup vs baseline: 2.4159x; 2.4159x over previous
import jax
import jax.numpy as jnp
from jax.experimental import pallas as pl
from jax.experimental.pallas import tpu as pltpu


def _sigmoid(x):
    return 0.5 * jnp.tanh(0.5 * x) + 0.5


def _ffn_body(x_ref, w13_ref, w2_ref, rms_ref, wf_ref, d_ref, wfb_ref):
    F = w2_ref.shape[0]
    x = x_ref[...].astype(jnp.bfloat16)
    a = jnp.dot(x, w13_ref[:, :F], preferred_element_type=jnp.float32)
    c = jnp.dot(x, w13_ref[:, F:], preferred_element_type=jnp.float32)
    h = (a * _sigmoid(a)) * c
    dec = jnp.dot(h.astype(jnp.bfloat16), w2_ref[...],
                  preferred_element_type=jnp.float32)
    dec = dec * _sigmoid(dec)
    dec = dec * jax.lax.rsqrt(
        jnp.mean(dec * dec, axis=-1, keepdims=True) + 1e-6)
    dec = dec * rms_ref[...]
    d_ref[...] = dec.astype(jnp.bfloat16)
    wfb_ref[...] = wf_ref[...].astype(jnp.bfloat16)


def _out_body(d_ref, wfb_ref, bias_ref, out_ref):
    ob = pl.program_id(1)
    BO = out_ref.shape[1]
    logit = jnp.dot(d_ref[...], wfb_ref[:, pl.ds(ob * BO, BO)],
                    preferred_element_type=jnp.float32)
    logit = logit + bias_ref[:, pl.ds(ob * BO, BO)]
    out_ref[...] = _sigmoid(logit)


@jax.jit
def _run(x, W1, W2, W3, rms_w, Wf, bf):
    L, D = x.shape
    F = W1.shape[1]
    O = Wf.shape[1]

    w13b = jnp.concatenate(
        [W1.astype(jnp.bfloat16), W3.astype(jnp.bfloat16)], axis=1)
    w2b = W2.astype(jnp.bfloat16)
    rms2 = rms_w.reshape(1, D)
    bf2 = bf.reshape(1, O)

    BL_A = min(256, L)
    grid_a = L // BL_A
    WFC = O // grid_a

    d, wfb = pl.pallas_call(
        _ffn_body,
        grid=(grid_a,),
        in_specs=[
            pl.BlockSpec((BL_A, D), lambda i: (i, 0)),
            pl.BlockSpec((D, 2 * F), lambda i: (0, 0)),
            pl.BlockSpec((F, D), lambda i: (0, 0)),
            pl.BlockSpec((1, D), lambda i: (0, 0)),
            pl.BlockSpec((D, WFC), lambda i: (0, i)),
        ],
        out_specs=[
            pl.BlockSpec((BL_A, D), lambda i: (i, 0)),
            pl.BlockSpec((D, WFC), lambda i: (0, i)),
        ],
        out_shape=[
            jax.ShapeDtypeStruct((L, D), jnp.bfloat16),
            jax.ShapeDtypeStruct((D, O), jnp.bfloat16),
        ],
        compiler_params=pltpu.CompilerParams(
            dimension_semantics=("arbitrary",),
        ),
    )(x, w13b, w2b, rms2, Wf)

    BL_B = min(1024, L)
    BO_B = min(4096, O)
    out = pl.pallas_call(
        _out_body,
        grid=(L // BL_B, O // BO_B),
        in_specs=[
            pl.BlockSpec((BL_B, D), lambda lb, ob: (lb, 0)),
            pl.BlockSpec((D, O), lambda lb, ob: (0, 0)),
            pl.BlockSpec((1, O), lambda lb, ob: (0, 0)),
        ],
        out_specs=pl.BlockSpec((BL_B, BO_B), lambda lb, ob: (lb, ob)),
        out_shape=jax.ShapeDtypeStruct((L, O), jnp.float32),
        compiler_params=pltpu.CompilerParams(
            dimension_semantics=("arbitrary", "arbitrary"),
        ),
    )(d, wfb, bf2)
    return out


def kernel(inputs, label, W1, W2, W3, rms_w, Wf, bf):
    del label
    x = inputs[0]
    out = _run(x, W1, W2, W3, rms_w, Wf, bf)
    return out[None]


# speedup vs baseline: 2.4876x; 1.0297x over previous
import jax
import jax.numpy as jnp
from jax.experimental import pallas as pl
from jax.experimental.pallas import tpu as pltpu


def _sigmoid(x):
    return 0.5 * jnp.tanh(0.5 * x) + 0.5


def _ffn_body(x_ref, w13_ref, w2_ref, rms_ref, wf_ref, d_ref, wfb_ref,
              w2b_ref):
    F = w2_ref.shape[0]

    @pl.when(pl.program_id(0) == 0)
    def _cvt():
        w2b_ref[...] = w2_ref[...].astype(jnp.bfloat16)

    x = x_ref[...].astype(jnp.bfloat16)
    FH = F // 2
    dec = jnp.zeros((x.shape[0], x_ref.shape[1]), jnp.float32)
    for k in range(2):
        a = jnp.dot(x, w13_ref[:, k * FH:(k + 1) * FH],
                    preferred_element_type=jnp.float32)
        c = jnp.dot(x, w13_ref[:, F + k * FH:F + (k + 1) * FH],
                    preferred_element_type=jnp.float32)
        h = (a * _sigmoid(a)) * c
        dec = dec + jnp.dot(h.astype(jnp.bfloat16),
                            w2b_ref[pl.ds(k * FH, FH), :],
                            preferred_element_type=jnp.float32)
    dec = dec * _sigmoid(dec)
    dec = dec * jax.lax.rsqrt(
        jnp.mean(dec * dec, axis=-1, keepdims=True) + 1e-6)
    dec = dec * rms_ref[...]
    d_ref[...] = dec.astype(jnp.bfloat16)
    wfb_ref[...] = wf_ref[...].astype(jnp.bfloat16)


def _out_body(d_ref, wfb_ref, bias_ref, out_ref):
    ob = pl.program_id(1)
    BO = out_ref.shape[1]
    logit = jnp.dot(d_ref[...], wfb_ref[:, pl.ds(ob * BO, BO)],
                    preferred_element_type=jnp.float32)
    logit = logit + bias_ref[:, pl.ds(ob * BO, BO)]
    out_ref[...] = _sigmoid(logit)


@jax.jit
def _run(x, W1, W2, W3, rms_w, Wf, bf):
    L, D = x.shape
    F = W1.shape[1]
    O = Wf.shape[1]

    w13b = jnp.concatenate(
        [W1.astype(jnp.bfloat16), W3.astype(jnp.bfloat16)], axis=1)
    rms2 = rms_w.reshape(1, D)
    bf2 = bf.reshape(1, O)

    BL_A = min(256, L)
    grid_a = L // BL_A
    WFC = O // grid_a

    d, wfb = pl.pallas_call(
        _ffn_body,
        grid=(grid_a,),
        in_specs=[
            pl.BlockSpec((BL_A, D), lambda i: (i, 0)),
            pl.BlockSpec((D, 2 * F), lambda i: (0, 0)),
            pl.BlockSpec((F, D), lambda i: (0, 0)),
            pl.BlockSpec((1, D), lambda i: (0, 0)),
            pl.BlockSpec((D, WFC), lambda i: (0, i)),
        ],
        out_specs=[
            pl.BlockSpec((BL_A, D), lambda i: (i, 0)),
            pl.BlockSpec((D, WFC), lambda i: (0, i)),
        ],
        out_shape=[
            jax.ShapeDtypeStruct((L, D), jnp.bfloat16),
            jax.ShapeDtypeStruct((D, O), jnp.bfloat16),
        ],
        scratch_shapes=[pltpu.VMEM((F, D), jnp.bfloat16)],
        compiler_params=pltpu.CompilerParams(
            dimension_semantics=("arbitrary",),
        ),
    )(x, w13b, W2, rms2, Wf)

    BL_B = min(1024, L)
    BO_B = min(4096, O)
    out = pl.pallas_call(
        _out_body,
        grid=(L // BL_B, O // BO_B),
        in_specs=[
            pl.BlockSpec((BL_B, D), lambda lb, ob: (lb, 0)),
            pl.BlockSpec((D, O), lambda lb, ob: (0, 0)),
            pl.BlockSpec((1, O), lambda lb, ob: (0, 0)),
        ],
        out_specs=pl.BlockSpec((BL_B, BO_B), lambda lb, ob: (lb, ob)),
        out_shape=jax.ShapeDtypeStruct((L, O), jnp.float32),
        compiler_params=pltpu.CompilerParams(
            dimension_semantics=("arbitrary", "arbitrary"),
        ),
    )(d, wfb, bf2)
    return out


def kernel(inputs, label, W1, W2, W3, rms_w, Wf, bf):
    del label
    x = inputs[0]
    out = _run(x, W1, W2, W3, rms_w, Wf, bf)
    return out[None]


# speedup vs baseline: 2.4983x; 1.0043x over previous
import jax
import jax.numpy as jnp
from jax.experimental import pallas as pl
from jax.experimental.pallas import tpu as pltpu


def _sigmoid(x):
    return 0.5 * jnp.tanh(0.5 * x) + 0.5


def _ffn_body(x_ref, w1_ref, w3_ref, w2_ref, rms_ref, wf_ref, d_ref, wfb_ref,
              w2b_ref):
    F = w2_ref.shape[0]

    @pl.when(pl.program_id(0) == 0)
    def _cvt():
        w2b_ref[...] = w2_ref[...].astype(jnp.bfloat16)

    x = x_ref[...].astype(jnp.bfloat16)
    FH = F // 2
    dec = jnp.zeros((x.shape[0], x_ref.shape[1]), jnp.float32)
    for k in range(2):
        a = jnp.dot(x, w1_ref[:, k * FH:(k + 1) * FH],
                    preferred_element_type=jnp.float32)
        c = jnp.dot(x, w3_ref[:, k * FH:(k + 1) * FH],
                    preferred_element_type=jnp.float32)
        h = (a * _sigmoid(a)) * c
        dec = dec + jnp.dot(h.astype(jnp.bfloat16),
                            w2b_ref[pl.ds(k * FH, FH), :],
                            preferred_element_type=jnp.float32)
    dec = dec * _sigmoid(dec)
    dec = dec * jax.lax.rsqrt(
        jnp.mean(dec * dec, axis=-1, keepdims=True) + 1e-6)
    dec = dec * rms_ref[...]
    d_ref[...] = dec.astype(jnp.bfloat16)
    wfb_ref[...] = wf_ref[...].astype(jnp.bfloat16)


def _out_body(d_ref, wfb_ref, bias_ref, out_ref):
    ob = pl.program_id(1)
    BO = out_ref.shape[1]
    logit = jnp.dot(d_ref[...], wfb_ref[:, pl.ds(ob * BO, BO)],
                    preferred_element_type=jnp.float32)
    logit = logit + bias_ref[:, pl.ds(ob * BO, BO)]
    out_ref[...] = _sigmoid(logit)


@jax.jit
def _run(x, W1, W2, W3, rms_w, Wf, bf):
    L, D = x.shape
    F = W1.shape[1]
    O = Wf.shape[1]

    w1b = W1.astype(jnp.bfloat16)
    w3b = W3.astype(jnp.bfloat16)
    rms2 = rms_w.reshape(1, D)
    bf2 = bf.reshape(1, O)

    BL_A = min(256, L)
    grid_a = L // BL_A
    WFC = O // grid_a

    d, wfb = pl.pallas_call(
        _ffn_body,
        grid=(grid_a,),
        in_specs=[
            pl.BlockSpec((BL_A, D), lambda i: (i, 0)),
            pl.BlockSpec((D, F), lambda i: (0, 0)),
            pl.BlockSpec((D, F), lambda i: (0, 0)),
            pl.BlockSpec((F, D), lambda i: (0, 0)),
            pl.BlockSpec((1, D), lambda i: (0, 0)),
            pl.BlockSpec((D, WFC), lambda i: (0, i)),
        ],
        out_specs=[
            pl.BlockSpec((BL_A, D), lambda i: (i, 0)),
            pl.BlockSpec((D, WFC), lambda i: (0, i)),
        ],
        out_shape=[
            jax.ShapeDtypeStruct((L, D), jnp.bfloat16),
            jax.ShapeDtypeStruct((D, O), jnp.bfloat16),
        ],
        scratch_shapes=[pltpu.VMEM((F, D), jnp.bfloat16)],
        compiler_params=pltpu.CompilerParams(
            dimension_semantics=("arbitrary",),
        ),
    )(x, w1b, w3b, W2, rms2, Wf)

    BL_B = min(1024, L)
    BO_B = min(4096, O)
    out = pl.pallas_call(
        _out_body,
        grid=(L // BL_B, O // BO_B),
        in_specs=[
            pl.BlockSpec((BL_B, D), lambda lb, ob: (lb, 0)),
            pl.BlockSpec((D, O), lambda lb, ob: (0, 0)),
            pl.BlockSpec((1, O), lambda lb, ob: (0, 0)),
        ],
        out_specs=pl.BlockSpec((BL_B, BO_B), lambda lb, ob: (lb, ob)),
        out_shape=jax.ShapeDtypeStruct((L, O), jnp.float32),
        compiler_params=pltpu.CompilerParams(
            dimension_semantics=("arbitrary", "arbitrary"),
        ),
    )(d, wfb, bf2)
    return out


def kernel(inputs, label, W1, W2, W3, rms_w, Wf, bf):
    del label
    x = inputs[0]
    out = _run(x, W1, W2, W3, rms_w, Wf, bf)
    return out[None]
